# baseline (speedup 1.0000x reference)
import functools

import jax
import jax.numpy as jnp
from jax import lax
from jax.experimental import pallas as pl

_N = 5000
_K = 300
_SIGMA = 0.5
_BLK = 512
_NPAD = 5120
_NBLK = _NPAD // _BLK
_NEG = -1e30


def _corners(b_rows, b_cols):
    rx1 = jnp.minimum(b_rows[:, 0:1], b_rows[:, 2:3])
    ry1 = jnp.minimum(b_rows[:, 1:2], b_rows[:, 3:4])
    rx2 = jnp.maximum(b_rows[:, 0:1], b_rows[:, 2:3])
    ry2 = jnp.maximum(b_rows[:, 1:2], b_rows[:, 3:4])
    ra = (rx2 - rx1) * (ry2 - ry1)
    cx1 = jnp.minimum(b_cols[0:1, :], b_cols[2:3, :])
    cy1 = jnp.minimum(b_cols[1:2, :], b_cols[3:4, :])
    cx2 = jnp.maximum(b_cols[0:1, :], b_cols[2:3, :])
    cy2 = jnp.maximum(b_cols[1:2, :], b_cols[3:4, :])
    ca = (cx2 - cx1) * (cy2 - cy1)
    return (rx1, ry1, rx2, ry2, ra), (cx1, cy1, cx2, cy2, ca)


def _iou_tile(rows, cols):
    (rx1, ry1, rx2, ry2, ra), (cx1, cy1, cx2, cy2, ca) = rows, cols
    iw = jnp.maximum(jnp.minimum(rx2, cx2) - jnp.maximum(rx1, cx1), 0.0)
    ih = jnp.maximum(jnp.minimum(ry2, cy2) - jnp.maximum(ry1, cy1), 0.0)
    inter = iw * ih
    union = ra + ca - inter + 1e-8
    return inter / union


def _higher_mask(i, j, s_rows, s_cols):
    ridx = i * _BLK + lax.broadcasted_iota(jnp.int32, (_BLK, _BLK), 0)
    cidx = j * _BLK + lax.broadcasted_iota(jnp.int32, (_BLK, _BLK), 1)
    return (s_cols > s_rows) | ((s_cols == s_rows) & (cidx < ridx))


def _comp_body(br, bc, sr, sc, comp_ref):
    i = pl.program_id(0)
    j = pl.program_id(1)
    rows, cols = _corners(br[...], bc[...])
    iou = _iou_tile(rows, cols)
    m = _higher_mask(i, j, sr[...], sc[...])
    contrib = jnp.where(m, iou, 0.0)
    tile_max = jnp.max(contrib, axis=1, keepdims=True)
    acc = jnp.where(j == 0, tile_max, jnp.maximum(comp_ref[...], tile_max))
    comp_ref[...] = acc


def _decay_body(br, bc, sr, sc, cc, out_ref):
    i = pl.program_id(0)
    j = pl.program_id(1)
    nj = pl.num_programs(1)
    rows, cols = _corners(br[...], bc[...])
    iou = _iou_tile(rows, cols)
    m = _higher_mask(i, j, sr[...], sc[...])
    comp2 = cc[...] * cc[...]
    contrib = jnp.where(m, iou * iou - comp2, _NEG)
    tile_max = jnp.max(contrib, axis=1, keepdims=True)
    acc = jnp.where(j == 0, tile_max, jnp.maximum(out_ref[...], tile_max))

    @pl.when(j < nj - 1)
    def _():
        out_ref[...] = acc

    @pl.when(j == nj - 1)
    def _():
        ridx = i * _BLK + lax.broadcasted_iota(jnp.int32, (_BLK, 1), 0)
        new_s = sr[...] * jnp.exp(-jnp.maximum(acc, 0.0) / _SIGMA)
        out_ref[...] = jnp.where(ridx < _N, new_s, _NEG)


def _row_spec():
    return pl.BlockSpec((_BLK, 4), lambda i, j: (i, 0))


def _col_spec():
    return pl.BlockSpec((4, _BLK), lambda i, j: (0, j))


def _srow_spec():
    return pl.BlockSpec((_BLK, 1), lambda i, j: (i, 0))


def _scol_spec():
    return pl.BlockSpec((1, _BLK), lambda i, j: (0, j))


@jax.jit
def kernel(boxes, scores):
    pad = _NPAD - _N
    b = jnp.pad(boxes, ((0, pad), (0, 0)))
    s = jnp.pad(scores, (0, pad), constant_values=-1.0)
    bt = b.T
    s_row = s[:, None]
    s_col = s[None, :]

    grid = (_NBLK, _NBLK)
    comp = pl.pallas_call(
        _comp_body,
        grid=grid,
        in_specs=[_row_spec(), _col_spec(), _srow_spec(), _scol_spec()],
        out_specs=pl.BlockSpec((_BLK, 1), lambda i, j: (i, 0)),
        out_shape=jax.ShapeDtypeStruct((_NPAD, 1), jnp.float32),
    )(b, bt, s_row, s_col)

    new_s = pl.pallas_call(
        _decay_body,
        grid=grid,
        in_specs=[
            _row_spec(),
            _col_spec(),
            _srow_spec(),
            _scol_spec(),
            pl.BlockSpec((1, _BLK), lambda i, j: (0, j)),
        ],
        out_specs=pl.BlockSpec((_BLK, 1), lambda i, j: (i, 0)),
        out_shape=jax.ShapeDtypeStruct((_NPAD, 1), jnp.float32),
    )(b, bt, s_row, s_col, comp.T)

    vals, idx = lax.top_k(new_s[:, 0], _K)
    sel = jnp.take(boxes, idx, axis=0)
    return jnp.concatenate([sel, vals[:, None]], axis=1)


# speedup vs baseline: 1.5456x; 1.5456x over previous
import functools

import jax
import jax.numpy as jnp
from jax import lax
from jax.experimental import pallas as pl

_N = 5000
_K = 300
_SIGMA = 0.5
_BLK = 512
_NPAD = 5120
_NBLK = _NPAD // _BLK
_NEG = -1e30


def _row_slices(refs, i):
    return [r[pl.ds(i * _BLK, _BLK), :] for r in refs]


def _col_slices(refs, j):
    return [r[:, pl.ds(j * _BLK, _BLK)] for r in refs]


def _iou_tile(rows, cols):
    rx1, ry1, rx2, ry2, ra = rows
    cx1, cy1, cx2, cy2, ca = cols
    iw = jnp.maximum(jnp.minimum(rx2, cx2) - jnp.maximum(rx1, cx1), 0.0)
    ih = jnp.maximum(jnp.minimum(ry2, cy2) - jnp.maximum(ry1, cy1), 0.0)
    inter = iw * ih
    union = (ra + ca) - inter
    return inter / union


def _diag_mask(s_r, s_c):
    ridx = lax.broadcasted_iota(jnp.int32, (_BLK, _BLK), 0)
    cidx = lax.broadcasted_iota(jnp.int32, (_BLK, _BLK), 1)
    return (s_c > s_r) | ((s_c == s_r) & (cidx < ridx))


def _comp_body(x1r, y1r, x2r, y2r, ar, x1c, y1c, x2c, y2c, ac, sr, sc,
               comp_r_ref, comp_c_ref):
    rrefs = (x1r, y1r, x2r, y2r, ar)
    crefs = (x1c, y1c, x2c, y2c, ac)
    comp_c_ref[...] = jnp.zeros((1, _NPAD), jnp.float32)

    def outer(i, _):
        rows = _row_slices(rrefs, i)
        s_r = sr[pl.ds(i * _BLK, _BLK), :]

        def inner(j, acc):
            cols = _col_slices(crefs, j)
            s_c = sc[:, pl.ds(j * _BLK, _BLK)]
            iou = _iou_tile(rows, cols)
            m = s_c >= s_r
            acc = jnp.maximum(acc, jnp.max(jnp.where(m, iou, 0.0), axis=1,
                                           keepdims=True))
            cmax = jnp.max(jnp.where(m, 0.0, iou), axis=0, keepdims=True)
            sl = (slice(0, 1), pl.ds(j * _BLK, _BLK))
            comp_c_ref[sl] = jnp.maximum(comp_c_ref[sl], cmax)
            return acc

        acc = lax.fori_loop(0, i, inner, jnp.zeros((_BLK, 1), jnp.float32))
        cols = _col_slices(crefs, i)
        s_c = sc[:, pl.ds(i * _BLK, _BLK)]
        iou = _iou_tile(rows, cols)
        m = _diag_mask(s_r, s_c)
        acc = jnp.maximum(acc, jnp.max(jnp.where(m, iou, 0.0), axis=1,
                                       keepdims=True))
        comp_r_ref[pl.ds(i * _BLK, _BLK), :] = acc
        return 0

    lax.fori_loop(0, _NBLK, outer, 0)


def _decay_body(x1r, y1r, x2r, y2r, ar, x1c, y1c, x2c, y2c, ac, sr, sc,
                c2r, c2c, q_r_ref, q_c_ref):
    rrefs = (x1r, y1r, x2r, y2r, ar)
    crefs = (x1c, y1c, x2c, y2c, ac)
    q_c_ref[...] = jnp.full((1, _NPAD), _NEG, jnp.float32)

    def outer(i, _):
        rows = _row_slices(rrefs, i)
        s_r = sr[pl.ds(i * _BLK, _BLK), :]
        comp2_r = c2r[pl.ds(i * _BLK, _BLK), :]

        def inner(j, acc):
            cols = _col_slices(crefs, j)
            s_c = sc[:, pl.ds(j * _BLK, _BLK)]
            comp2_c = c2c[:, pl.ds(j * _BLK, _BLK)]
            iou = _iou_tile(rows, cols)
            iou2 = iou * iou
            m = s_c >= s_r
            acc = jnp.maximum(
                acc,
                jnp.max(jnp.where(m, iou2 - comp2_c, _NEG), axis=1,
                        keepdims=True))
            cmax = jnp.max(jnp.where(m, _NEG, iou2 - comp2_r), axis=0,
                           keepdims=True)
            sl = (slice(0, 1), pl.ds(j * _BLK, _BLK))
            q_c_ref[sl] = jnp.maximum(q_c_ref[sl], cmax)
            return acc

        acc = lax.fori_loop(0, i, inner, jnp.full((_BLK, 1), _NEG,
                                                  jnp.float32))
        cols = _col_slices(crefs, i)
        s_c = sc[:, pl.ds(i * _BLK, _BLK)]
        comp2_c = c2c[:, pl.ds(i * _BLK, _BLK)]
        iou = _iou_tile(rows, cols)
        iou2 = iou * iou
        m = _diag_mask(s_r, s_c)
        acc = jnp.maximum(
            acc,
            jnp.max(jnp.where(m, iou2 - comp2_c, _NEG), axis=1,
                    keepdims=True))
        q_r_ref[pl.ds(i * _BLK, _BLK), :] = acc
        return 0

    lax.fori_loop(0, _NBLK, outer, 0)


def _rescore_body(q_r, q_ct, sr, out_ref):
    ridx = lax.broadcasted_iota(jnp.int32, (_NPAD, 1), 0)
    q = jnp.maximum(jnp.maximum(q_r[...], q_ct[...]), 0.0)
    new_s = sr[...] * jnp.exp(-q / _SIGMA)
    out_ref[...] = jnp.where(ridx < _N, new_s, _NEG)


@jax.jit
def kernel(boxes, scores):
    pad = _NPAD - _N
    b = jnp.pad(boxes, ((0, pad), (0, 0)))
    s = jnp.pad(scores, (0, pad), constant_values=-1.0)

    x1 = jnp.minimum(b[:, 0], b[:, 2])
    y1 = jnp.minimum(b[:, 1], b[:, 3])
    x2 = jnp.maximum(b[:, 0], b[:, 2])
    y2 = jnp.maximum(b[:, 1], b[:, 3])
    area = (x2 - x1) * (y2 - y1) + 1e-8

    rowv = lambda v: v[:, None]
    colv = lambda v: v[None, :]
    row_args = [rowv(x1), rowv(y1), rowv(x2), rowv(y2), rowv(area)]
    col_args = [colv(x1), colv(y1), colv(x2), colv(y2), colv(area)]
    s_row, s_col = rowv(s), colv(s)

    f32 = jnp.float32
    comp_r, comp_c = pl.pallas_call(
        _comp_body,
        out_shape=(jax.ShapeDtypeStruct((_NPAD, 1), f32),
                   jax.ShapeDtypeStruct((1, _NPAD), f32)),
    )(*row_args, *col_args, s_row, s_col)

    comp = jnp.maximum(comp_r[:, 0], comp_c[0, :])
    comp2 = comp * comp
    q_r, q_c = pl.pallas_call(
        _decay_body,
        out_shape=(jax.ShapeDtypeStruct((_NPAD, 1), f32),
                   jax.ShapeDtypeStruct((1, _NPAD), f32)),
    )(*row_args, *col_args, s_row, s_col, rowv(comp2), colv(comp2))

    new_s = pl.pallas_call(
        _rescore_body,
        out_shape=jax.ShapeDtypeStruct((_NPAD, 1), f32),
    )(q_r, q_c.T, s_row)

    vals, idx = lax.top_k(new_s[:, 0], _K)
    sel = jnp.take(boxes, idx, axis=0)
    return jnp.concatenate([sel, vals[:, None]], axis=1)
